# baseline (speedup 1.0000x reference)
import functools

import jax
import jax.numpy as jnp
from jax import lax
from jax.experimental import pallas as pl
from jax.experimental.pallas import tpu as pltpu
from jax.experimental.pallas import tpu_sc as plsc

_NB = 5


def _make_gather(V, D, B, NC, NS):
    NW = NC * NS
    C = 128
    b_per_w = B // NW
    K = b_per_w // C
    assert b_per_w * NW == B and K * C == b_per_w and K % _NB == 0

    mesh = plsc.VectorSubcoreMesh(core_axis_name="c", subcore_axis_name="s")

    @functools.partial(
        pl.kernel,
        mesh=mesh,
        out_type=jax.ShapeDtypeStruct((B, 2 * D), jnp.float32),
        scratch_types=[
            pltpu.VMEM((K, C), jnp.int32),
            [pltpu.VMEM((C, 2 * D), jnp.float32) for _ in range(_NB)],
            [pltpu.SemaphoreType.DMA for _ in range(_NB)],
            [pltpu.SemaphoreType.DMA for _ in range(_NB)],
        ],
    )
    def gather_kernel(r2d_hbm, idxp_hbm, out_hbm, idxp_v, bufs, gsems, wsems):
        wid = lax.axis_index("s") * NC + lax.axis_index("c")
        base = wid * b_per_w
        pltpu.sync_copy(idxp_hbm.at[wid], idxp_v)

        def gather(g, b):
            return pltpu.make_async_copy(
                r2d_hbm.at[idxp_v.at[g]], bufs[b], gsems[b])

        def writeback(g, b):
            return pltpu.make_async_copy(
                bufs[b], out_hbm.at[pl.ds(base + g * C, C)], wsems[b])

        for b in range(_NB - 1):
            gather(b, b).start()

        @pl.loop(0, K, step=_NB)
        def _lap(j):
            for b in range(_NB):
                g = j + b
                bp = (b - 1) % _NB

                if b == 0:
                    @pl.when(j >= 1)
                    def _wbwait0():
                        writeback(g - 1, bp).wait()
                else:
                    writeback(g - 1, bp).wait()
                @pl.when(g + _NB - 1 < K)
                def _refill():
                    gather(g + _NB - 1, bp).start()
                gather(g, b).wait()
                writeback(g, b).start()

        writeback(K - 1, (K - 1) % _NB).wait()

    return gather_kernel


def kernel(input, dim, index):
    table = input
    V, D = table.shape
    (B,) = index.shape
    info = plsc.get_sparse_core_info()
    NC, NS = info.num_cores, info.num_subcores
    NW = NC * NS
    C = 128
    idx = index.astype(jnp.int32)
    idxp = (idx // 2).reshape(NW, (B // NW) // C, C)
    r2d = table.reshape(V // 2, 2 * D)
    pairs = _make_gather(V, D, B, NC, NS)(r2d, idxp)
    hi = (idx % 2 == 1)[:, None]
    return jnp.where(hi, pairs[:, D:], pairs[:, :D])
